# Initial kernel scaffold; baseline (speedup 1.0000x reference)
import jax
import jax.numpy as jnp
from jax.experimental import pallas as pl


def kernel(c_seq, d_seq, r_seq, D1, D2, v_r, v_beta, W_ih, W_hh, b_ih, b_hh, W1a, b1a, W1b, b1b, W1c, b1c, W2a, b2a, W2b, b2b, W2c, b2c):
    raise NotImplementedError("write your pallas kernel here")


# speedup vs baseline: 9.8569x; 9.8569x over previous
import jax
import jax.numpy as jnp
from jax.experimental import pallas as pl
from jax.experimental.pallas import tpu as pltpu

B = 128
S = 200
NUM_C = 512
NUM_D = 128
DIM_V = 256


def _prep_body(d2, w_ihd_t, w_ihr_t, v_r2, w2ad_t, w2ar_t, w2ab_t, v_beta2,
               g_gi, w_gir, g2, w2r, u2):
    f32 = jnp.float32
    g_gi[...] = jnp.dot(d2[...], w_ihd_t[...], preferred_element_type=f32)
    w_gir[...] = jnp.dot(v_r2[...], w_ihr_t[...], preferred_element_type=f32)
    g2[...] = jnp.dot(d2[...], w2ad_t[...], preferred_element_type=f32)
    w2r[...] = jnp.dot(v_r2[...], w2ar_t[...], preferred_element_type=f32)
    u2[...] = jnp.dot(v_beta2[...], w2ab_t[...], preferred_element_type=f32)


def _step_body(c3, d3, r3,
               g_gi, w_gir, b_ih2, w_hh_t, b_hh2,
               w1a_t, b1a2, w1b_t, b1b2, w1c_t, b1c2,
               g2, w2r, u2, b2a2, w2b_t, b2b2, w2c_t, b2c2, d1,
               alpha_o, beta_o, gamma_o, h_o, c_o,
               h_scr, c_scr):
    f32 = jnp.float32
    t = pl.program_id(0)

    @pl.when(t == 0)
    def _():
        h_scr[...] = jnp.zeros_like(h_scr)
        c_scr[...] = jnp.zeros_like(c_scr)

    c = c3[0]
    d = d3[0]
    r = r3[0]

    oh_d = (d == jax.lax.broadcasted_iota(jnp.int32, (B, NUM_D), 1)).astype(f32)
    oh_c = c == jax.lax.broadcasted_iota(jnp.int32, (B, NUM_C), 1)

    h = h_scr[...]
    gi = (jnp.dot(oh_d, g_gi[...], preferred_element_type=f32)
          + r * w_gir[...] + b_ih2[...])
    gh = jnp.dot(h, w_hh_t[...], preferred_element_type=f32) + b_hh2[...]
    i_r, i_z, i_n = gi[:, :DIM_V], gi[:, DIM_V:2 * DIM_V], gi[:, 2 * DIM_V:]
    h_r, h_z, h_n = gh[:, :DIM_V], gh[:, DIM_V:2 * DIM_V], gh[:, 2 * DIM_V:]
    rg = jax.nn.sigmoid(i_r + h_r)
    z = jax.nn.sigmoid(i_z + h_z)
    n = jnp.tanh(i_n + rg * h_n)
    h_new = (1.0 - z) * n + z * h
    h_scr[...] = h_new
    h_o[:, jax.lax.rem(t, 8), :] = h_new

    a1 = jnp.maximum(jnp.dot(h_new, w1a_t[...], preferred_element_type=f32)
                     + b1a2[...], 0.0)
    a2 = jnp.maximum(jnp.dot(a1, w1b_t[...], preferred_element_type=f32)
                     + b1b2[...], 0.0)
    alpha_o[0] = jnp.dot(a2, w1c_t[...], preferred_element_type=f32) + b1c2[...]

    gamma_o[0] = jnp.dot(oh_d, d1[...], preferred_element_type=f32)

    cm = c_scr[...]
    bp = jnp.sum(jnp.where(oh_c, cm, 0.0), axis=1, keepdims=True)
    m1 = jnp.maximum(jnp.dot(oh_d, g2[...], preferred_element_type=f32)
                     + r * w2r[...] + bp * u2[...] + b2a2[...], 0.0)
    m2 = jnp.maximum(jnp.dot(m1, w2b_t[...], preferred_element_type=f32)
                     + b2b2[...], 0.0)
    new_c = jnp.dot(m2, w2c_t[...], preferred_element_type=f32) + b2c2[...]
    beta_o[0] = new_c
    cm_new = jnp.where(oh_c, new_c, cm)
    c_scr[...] = cm_new
    c_o[:, jax.lax.rem(t, 8), :] = cm_new


def kernel(c_seq, d_seq, r_seq, D1, D2, v_r, v_beta, W_ih, W_hh, b_ih, b_hh,
           W1a, b1a, W1b, b1b, W1c, b1c, W2a, b2a, W2b, b2b, W2c, b2c):
    f32 = jnp.float32
    c3 = c_seq.astype(jnp.int32).T.reshape(S, B, 1)
    d3 = d_seq.astype(jnp.int32).T.reshape(S, B, 1)
    r3 = r_seq.T.reshape(S, B, 1)
    v_r2 = v_r.reshape(1, DIM_V)
    v_beta2 = v_beta.reshape(1, DIM_V)

    g_gi, w_gir, g2, w2r, u2 = pl.pallas_call(
        _prep_body,
        out_shape=[
            jax.ShapeDtypeStruct((NUM_D, 3 * DIM_V), f32),
            jax.ShapeDtypeStruct((1, 3 * DIM_V), f32),
            jax.ShapeDtypeStruct((NUM_D, DIM_V), f32),
            jax.ShapeDtypeStruct((1, DIM_V), f32),
            jax.ShapeDtypeStruct((1, DIM_V), f32),
        ],
    )(D2, W_ih[:, :DIM_V].T, W_ih[:, DIM_V:].T, v_r2,
      W2a[:, DIM_V:2 * DIM_V].T, W2a[:, 2 * DIM_V:].T, W2a[:, :DIM_V].T,
      v_beta2)

    full = lambda shape: pl.BlockSpec(shape, lambda t: tuple(0 for _ in shape))
    in_specs = [
        pl.BlockSpec((1, B, 1), lambda t: (t, 0, 0)),
        pl.BlockSpec((1, B, 1), lambda t: (t, 0, 0)),
        pl.BlockSpec((1, B, 1), lambda t: (t, 0, 0)),
        full((NUM_D, 3 * DIM_V)),
        full((1, 3 * DIM_V)),
        full((1, 3 * DIM_V)),
        full((DIM_V, 3 * DIM_V)),
        full((1, 3 * DIM_V)),
        full((DIM_V, DIM_V)),
        full((1, DIM_V)),
        full((DIM_V, DIM_V)),
        full((1, DIM_V)),
        full((DIM_V, 1)),
        full((1, 1)),
        full((NUM_D, DIM_V)),
        full((1, DIM_V)),
        full((1, DIM_V)),
        full((1, DIM_V)),
        full((DIM_V, DIM_V)),
        full((1, DIM_V)),
        full((DIM_V, 1)),
        full((1, 1)),
        full((NUM_D, 1)),
    ]
    out_specs = [
        pl.BlockSpec((1, B, 1), lambda t: (t, 0, 0)),
        pl.BlockSpec((1, B, 1), lambda t: (t, 0, 0)),
        pl.BlockSpec((1, B, 1), lambda t: (t, 0, 0)),
        pl.BlockSpec((B, 8, DIM_V), lambda t: (0, t // 8, 0)),
        pl.BlockSpec((B, 8, NUM_C), lambda t: (0, t // 8, 0)),
    ]
    out_shape = [
        jax.ShapeDtypeStruct((S, B, 1), f32),
        jax.ShapeDtypeStruct((S, B, 1), f32),
        jax.ShapeDtypeStruct((S, B, 1), f32),
        jax.ShapeDtypeStruct((B, S, DIM_V), f32),
        jax.ShapeDtypeStruct((B, S, NUM_C), f32),
    ]
    alpha, beta, gamma, h_seq, c_out = pl.pallas_call(
        _step_body,
        grid=(S,),
        in_specs=in_specs,
        out_specs=out_specs,
        out_shape=out_shape,
        scratch_shapes=[
            pltpu.VMEM((B, DIM_V), f32),
            pltpu.VMEM((B, NUM_C), f32),
        ],
    )(c3, d3, r3,
      g_gi, w_gir, b_ih.reshape(1, -1), W_hh.T, b_hh.reshape(1, -1),
      W1a.T, b1a.reshape(1, -1), W1b.T, b1b.reshape(1, -1),
      W1c.T, b1c.reshape(1, 1),
      g2, w2r, u2, b2a.reshape(1, -1), W2b.T, b2b.reshape(1, -1),
      W2c.T, b2c.reshape(1, 1), D1)
    return (alpha[:, :, 0].T, beta[:, :, 0].T, gamma[:, :, 0].T, h_seq,
            c_out.reshape(B, S, NUM_C, 1))


# speedup vs baseline: 11.8674x; 1.2040x over previous
import jax
import jax.numpy as jnp
from jax.experimental import pallas as pl
from jax.experimental.pallas import tpu as pltpu

B = 128
S = 200
NUM_C = 512
NUM_D = 128
DIM_V = 256
U = 8


def _prep_body(d2, w_ihd_t, w_ihr_t, v_r2, w2ad_t, w2ar_t, w2ab_t, v_beta2,
               g_gi, w_gir, g2, w2r, u2):
    f32 = jnp.float32
    g_gi[...] = jnp.dot(d2[...], w_ihd_t[...], preferred_element_type=f32)
    w_gir[...] = jnp.dot(v_r2[...], w_ihr_t[...], preferred_element_type=f32)
    g2[...] = jnp.dot(d2[...], w2ad_t[...], preferred_element_type=f32)
    w2r[...] = jnp.dot(v_r2[...], w2ar_t[...], preferred_element_type=f32)
    u2[...] = jnp.dot(v_beta2[...], w2ab_t[...], preferred_element_type=f32)


def _step_body(c3, d3, r3,
               g_gi, w_gir, b_ih2, w_hh_t, b_hh2,
               w1a_t, b1a2, w1b_t, b1b2, w1c_t, b1c2,
               g2, w2r, u2, b2a2, w2b_t, b2b2, w2c_t, b2c2, d1,
               alpha_o, beta_o, gamma_o, h_o, c_o,
               h_scr, c_scr):
    f32 = jnp.float32
    g = pl.program_id(0)

    @pl.when(g == 0)
    def _():
        h_scr[...] = jnp.zeros_like(h_scr)
        c_scr[...] = jnp.zeros_like(c_scr)

    iota_d = jax.lax.broadcasted_iota(jnp.int32, (B, NUM_D), 1)
    iota_c = jax.lax.broadcasted_iota(jnp.int32, (B, NUM_C), 1)

    h = h_scr[...]
    cm = c_scr[...]
    for j in range(U):
        c = c3[j]
        d = d3[j]
        r = r3[j]

        oh_d = (d == iota_d).astype(f32)
        oh_c = c == iota_c

        gi = (jnp.dot(oh_d, g_gi[...], preferred_element_type=f32)
              + r * w_gir[...] + b_ih2[...])
        gh = jnp.dot(h, w_hh_t[...], preferred_element_type=f32) + b_hh2[...]
        i_r, i_z, i_n = gi[:, :DIM_V], gi[:, DIM_V:2 * DIM_V], gi[:, 2 * DIM_V:]
        h_r, h_z, h_n = gh[:, :DIM_V], gh[:, DIM_V:2 * DIM_V], gh[:, 2 * DIM_V:]
        rg = jax.nn.sigmoid(i_r + h_r)
        z = jax.nn.sigmoid(i_z + h_z)
        n = jnp.tanh(i_n + rg * h_n)
        h = (1.0 - z) * n + z * h
        h_o[:, j, :] = h

        a1 = jnp.maximum(jnp.dot(h, w1a_t[...], preferred_element_type=f32)
                         + b1a2[...], 0.0)
        a2 = jnp.maximum(jnp.dot(a1, w1b_t[...], preferred_element_type=f32)
                         + b1b2[...], 0.0)
        alpha_o[j] = jnp.dot(a2, w1c_t[...], preferred_element_type=f32) + b1c2[...]

        gamma_o[j] = jnp.dot(oh_d, d1[...], preferred_element_type=f32)

        bp = jnp.sum(jnp.where(oh_c, cm, 0.0), axis=1, keepdims=True)
        m1 = jnp.maximum(jnp.dot(oh_d, g2[...], preferred_element_type=f32)
                         + r * w2r[...] + bp * u2[...] + b2a2[...], 0.0)
        m2 = jnp.maximum(jnp.dot(m1, w2b_t[...], preferred_element_type=f32)
                         + b2b2[...], 0.0)
        new_c = jnp.dot(m2, w2c_t[...], preferred_element_type=f32) + b2c2[...]
        beta_o[j] = new_c
        cm = jnp.where(oh_c, new_c, cm)
        c_o[:, j, :] = cm
    h_scr[...] = h
    c_scr[...] = cm


def kernel(c_seq, d_seq, r_seq, D1, D2, v_r, v_beta, W_ih, W_hh, b_ih, b_hh,
           W1a, b1a, W1b, b1b, W1c, b1c, W2a, b2a, W2b, b2b, W2c, b2c):
    f32 = jnp.float32
    c3 = c_seq.astype(jnp.int32).T.reshape(S, B, 1)
    d3 = d_seq.astype(jnp.int32).T.reshape(S, B, 1)
    r3 = r_seq.T.reshape(S, B, 1)
    v_r2 = v_r.reshape(1, DIM_V)
    v_beta2 = v_beta.reshape(1, DIM_V)

    g_gi, w_gir, g2, w2r, u2 = pl.pallas_call(
        _prep_body,
        out_shape=[
            jax.ShapeDtypeStruct((NUM_D, 3 * DIM_V), f32),
            jax.ShapeDtypeStruct((1, 3 * DIM_V), f32),
            jax.ShapeDtypeStruct((NUM_D, DIM_V), f32),
            jax.ShapeDtypeStruct((1, DIM_V), f32),
            jax.ShapeDtypeStruct((1, DIM_V), f32),
        ],
    )(D2, W_ih[:, :DIM_V].T, W_ih[:, DIM_V:].T, v_r2,
      W2a[:, DIM_V:2 * DIM_V].T, W2a[:, 2 * DIM_V:].T, W2a[:, :DIM_V].T,
      v_beta2)

    full = lambda shape: pl.BlockSpec(shape, lambda g: tuple(0 for _ in shape))
    in_specs = [
        pl.BlockSpec((U, B, 1), lambda g: (g, 0, 0)),
        pl.BlockSpec((U, B, 1), lambda g: (g, 0, 0)),
        pl.BlockSpec((U, B, 1), lambda g: (g, 0, 0)),
        full((NUM_D, 3 * DIM_V)),
        full((1, 3 * DIM_V)),
        full((1, 3 * DIM_V)),
        full((DIM_V, 3 * DIM_V)),
        full((1, 3 * DIM_V)),
        full((DIM_V, DIM_V)),
        full((1, DIM_V)),
        full((DIM_V, DIM_V)),
        full((1, DIM_V)),
        full((DIM_V, 1)),
        full((1, 1)),
        full((NUM_D, DIM_V)),
        full((1, DIM_V)),
        full((1, DIM_V)),
        full((1, DIM_V)),
        full((DIM_V, DIM_V)),
        full((1, DIM_V)),
        full((DIM_V, 1)),
        full((1, 1)),
        full((NUM_D, 1)),
    ]
    out_specs = [
        pl.BlockSpec((U, B, 1), lambda g: (g, 0, 0)),
        pl.BlockSpec((U, B, 1), lambda g: (g, 0, 0)),
        pl.BlockSpec((U, B, 1), lambda g: (g, 0, 0)),
        pl.BlockSpec((B, U, DIM_V), lambda g: (0, g, 0)),
        pl.BlockSpec((B, U, NUM_C), lambda g: (0, g, 0)),
    ]
    out_shape = [
        jax.ShapeDtypeStruct((S, B, 1), f32),
        jax.ShapeDtypeStruct((S, B, 1), f32),
        jax.ShapeDtypeStruct((S, B, 1), f32),
        jax.ShapeDtypeStruct((B, S, DIM_V), f32),
        jax.ShapeDtypeStruct((B, S, NUM_C), f32),
    ]
    alpha, beta, gamma, h_seq, c_out = pl.pallas_call(
        _step_body,
        grid=(S // U,),
        in_specs=in_specs,
        out_specs=out_specs,
        out_shape=out_shape,
        scratch_shapes=[
            pltpu.VMEM((B, DIM_V), f32),
            pltpu.VMEM((B, NUM_C), f32),
        ],
    )(c3, d3, r3,
      g_gi, w_gir, b_ih.reshape(1, -1), W_hh.T, b_hh.reshape(1, -1),
      W1a.T, b1a.reshape(1, -1), W1b.T, b1b.reshape(1, -1),
      W1c.T, b1c.reshape(1, 1),
      g2, w2r, u2, b2a.reshape(1, -1), W2b.T, b2b.reshape(1, -1),
      W2c.T, b2c.reshape(1, 1), D1)
    return (alpha[:, :, 0].T, beta[:, :, 0].T, gamma[:, :, 0].T, h_seq,
            c_out.reshape(B, S, NUM_C, 1))


# speedup vs baseline: 13.1662x; 1.1094x over previous
import jax
import jax.numpy as jnp
from jax.experimental import pallas as pl
from jax.experimental.pallas import tpu as pltpu

B = 128
S = 200
NUM_C = 512
NUM_D = 128
DIM_V = 256
U = 8
UB = U * B


def _prep_body(d2, w_ihd_t, w_ihr_t, v_r2, w2ad_t, w2ar_t, w2ab_t, v_beta2,
               g_gi, w_gir, g2, w2r, u2):
    f32 = jnp.float32
    g_gi[...] = jnp.dot(d2[...], w_ihd_t[...], preferred_element_type=f32)
    w_gir[...] = jnp.dot(v_r2[...], w_ihr_t[...], preferred_element_type=f32)
    g2[...] = jnp.dot(d2[...], w2ad_t[...], preferred_element_type=f32)
    w2r[...] = jnp.dot(v_r2[...], w2ar_t[...], preferred_element_type=f32)
    u2[...] = jnp.dot(v_beta2[...], w2ab_t[...], preferred_element_type=f32)


def _step_body(c3, d3, r3,
               g_gi, w_gir, b_ih2, w_hh_t, b_hh2,
               w1a_t, b1a2, w1b_t, b1b2, w1c2, b1c2,
               g2, w2r, u2, b2a2, w2b_t, b2b2, w2c2, b2c2, d1,
               alpha_o, beta_o, gamma_o, h_o, c_o,
               h_scr, c_scr, gi_scr, m1b_scr, h8_scr, beta8_scr):
    f32 = jnp.float32
    g = pl.program_id(0)

    @pl.when(g == 0)
    def _():
        h_scr[...] = jnp.zeros_like(h_scr)
        c_scr[...] = jnp.zeros_like(c_scr)

    d = d3[0]
    r = r3[0]
    iota_d = jax.lax.broadcasted_iota(jnp.int32, (UB, NUM_D), 1)
    iota_c = jax.lax.broadcasted_iota(jnp.int32, (B, NUM_C), 1)

    oh_d = (d == iota_d).astype(f32)
    gi_scr[...] = (jnp.dot(oh_d, g_gi[...], preferred_element_type=f32)
                   + r * w_gir[...] + b_ih2[...])
    m1b_scr[...] = (jnp.dot(oh_d, g2[...], preferred_element_type=f32)
                    + r * w2r[...] + b2a2[...])
    gamma_o[0] = jnp.dot(oh_d, d1[...], preferred_element_type=f32)

    h = h_scr[...]
    cm = c_scr[...]
    for j in range(U):
        c = c3[0, j * B:(j + 1) * B]
        oh_c = c == iota_c

        gi = gi_scr[j * B:(j + 1) * B, :]
        gh = jnp.dot(h, w_hh_t[...], preferred_element_type=f32) + b_hh2[...]
        rg = jax.nn.sigmoid(gi[:, :DIM_V] + gh[:, :DIM_V])
        z = jax.nn.sigmoid(gi[:, DIM_V:2 * DIM_V] + gh[:, DIM_V:2 * DIM_V])
        n = jnp.tanh(gi[:, 2 * DIM_V:] + rg * gh[:, 2 * DIM_V:])
        h = (1.0 - z) * n + z * h
        h_o[:, j, :] = h
        h8_scr[j * B:(j + 1) * B, :] = h

        bp = jnp.sum(jnp.where(oh_c, cm, 0.0), axis=1, keepdims=True)
        m1 = jnp.maximum(m1b_scr[j * B:(j + 1) * B, :] + bp * u2[...], 0.0)
        m2 = jnp.maximum(jnp.dot(m1, w2b_t[...], preferred_element_type=f32)
                         + b2b2[...], 0.0)
        new_c = jnp.sum(m2 * w2c2[...], axis=1, keepdims=True) + b2c2[...]
        beta8_scr[j * B:(j + 1) * B, :] = new_c
        cm = jnp.where(oh_c, new_c, cm)
        c_o[:, j, :] = cm
    h_scr[...] = h
    c_scr[...] = cm

    h8 = h8_scr[...]
    a1 = jnp.maximum(jnp.dot(h8, w1a_t[...], preferred_element_type=f32)
                     + b1a2[...], 0.0)
    a2 = jnp.maximum(jnp.dot(a1, w1b_t[...], preferred_element_type=f32)
                     + b1b2[...], 0.0)
    alpha_o[0] = jnp.sum(a2 * w1c2[...], axis=1, keepdims=True) + b1c2[...]
    beta_o[0] = beta8_scr[...]


def kernel(c_seq, d_seq, r_seq, D1, D2, v_r, v_beta, W_ih, W_hh, b_ih, b_hh,
           W1a, b1a, W1b, b1b, W1c, b1c, W2a, b2a, W2b, b2b, W2c, b2c):
    f32 = jnp.float32
    c3 = c_seq.astype(jnp.int32).T.reshape(S // U, UB, 1)
    d3 = d_seq.astype(jnp.int32).T.reshape(S // U, UB, 1)
    r3 = r_seq.T.reshape(S // U, UB, 1)
    v_r2 = v_r.reshape(1, DIM_V)
    v_beta2 = v_beta.reshape(1, DIM_V)

    g_gi, w_gir, g2, w2r, u2 = pl.pallas_call(
        _prep_body,
        out_shape=[
            jax.ShapeDtypeStruct((NUM_D, 3 * DIM_V), f32),
            jax.ShapeDtypeStruct((1, 3 * DIM_V), f32),
            jax.ShapeDtypeStruct((NUM_D, DIM_V), f32),
            jax.ShapeDtypeStruct((1, DIM_V), f32),
            jax.ShapeDtypeStruct((1, DIM_V), f32),
        ],
    )(D2, W_ih[:, :DIM_V].T, W_ih[:, DIM_V:].T, v_r2,
      W2a[:, DIM_V:2 * DIM_V].T, W2a[:, 2 * DIM_V:].T, W2a[:, :DIM_V].T,
      v_beta2)

    full = lambda shape: pl.BlockSpec(shape, lambda g: tuple(0 for _ in shape))
    in_specs = [
        pl.BlockSpec((1, UB, 1), lambda g: (g, 0, 0)),
        pl.BlockSpec((1, UB, 1), lambda g: (g, 0, 0)),
        pl.BlockSpec((1, UB, 1), lambda g: (g, 0, 0)),
        full((NUM_D, 3 * DIM_V)),
        full((1, 3 * DIM_V)),
        full((1, 3 * DIM_V)),
        full((DIM_V, 3 * DIM_V)),
        full((1, 3 * DIM_V)),
        full((DIM_V, DIM_V)),
        full((1, DIM_V)),
        full((DIM_V, DIM_V)),
        full((1, DIM_V)),
        full((1, DIM_V)),
        full((1, 1)),
        full((NUM_D, DIM_V)),
        full((1, DIM_V)),
        full((1, DIM_V)),
        full((1, DIM_V)),
        full((DIM_V, DIM_V)),
        full((1, DIM_V)),
        full((1, DIM_V)),
        full((1, 1)),
        full((NUM_D, 1)),
    ]
    out_specs = [
        pl.BlockSpec((1, UB, 1), lambda g: (g, 0, 0)),
        pl.BlockSpec((1, UB, 1), lambda g: (g, 0, 0)),
        pl.BlockSpec((1, UB, 1), lambda g: (g, 0, 0)),
        pl.BlockSpec((B, U, DIM_V), lambda g: (0, g, 0)),
        pl.BlockSpec((B, U, NUM_C), lambda g: (0, g, 0)),
    ]
    out_shape = [
        jax.ShapeDtypeStruct((S // U, UB, 1), f32),
        jax.ShapeDtypeStruct((S // U, UB, 1), f32),
        jax.ShapeDtypeStruct((S // U, UB, 1), f32),
        jax.ShapeDtypeStruct((B, S, DIM_V), f32),
        jax.ShapeDtypeStruct((B, S, NUM_C), f32),
    ]
    alpha, beta, gamma, h_seq, c_out = pl.pallas_call(
        _step_body,
        grid=(S // U,),
        in_specs=in_specs,
        out_specs=out_specs,
        out_shape=out_shape,
        scratch_shapes=[
            pltpu.VMEM((B, DIM_V), f32),
            pltpu.VMEM((B, NUM_C), f32),
            pltpu.VMEM((UB, 3 * DIM_V), f32),
            pltpu.VMEM((UB, DIM_V), f32),
            pltpu.VMEM((UB, DIM_V), f32),
            pltpu.VMEM((UB, 1), f32),
        ],
    )(c3, d3, r3,
      g_gi, w_gir, b_ih.reshape(1, -1), W_hh.T, b_hh.reshape(1, -1),
      W1a.T, b1a.reshape(1, -1), W1b.T, b1b.reshape(1, -1),
      W1c.reshape(1, DIM_V), b1c.reshape(1, 1),
      g2, w2r, u2, b2a.reshape(1, -1), W2b.T, b2b.reshape(1, -1),
      W2c.reshape(1, DIM_V), b2c.reshape(1, 1), D1)
    sb = lambda x: x.reshape(S, B).T
    return (sb(alpha), sb(beta), sb(gamma), h_seq,
            c_out.reshape(B, S, NUM_C, 1))


# speedup vs baseline: 14.1653x; 1.0759x over previous
import functools

import jax
import jax.numpy as jnp
from jax import lax
from jax.experimental import pallas as pl
from jax.experimental.pallas import tpu as pltpu
from jax.experimental.pallas import tpu_sc as plsc

B = 128
S = 200
NUM_C = 512
NUM_D = 128
DIM_V = 256
U = 8
UB = U * B

_NC = 2
_NS = 16
_NW = _NC * _NS
_RPW = B // _NW
_L = 16


def _prep_body(d2, w_ihd_t, w_ihr_t, v_r2, w2ad_t, w2ar_t, w2ab_t, v_beta2,
               g_gi, w_gir, g2, w2r, u2):
    f32 = jnp.float32
    g_gi[...] = jnp.dot(d2[...], w_ihd_t[...], preferred_element_type=f32)
    w_gir[...] = jnp.dot(v_r2[...], w_ihr_t[...], preferred_element_type=f32)
    g2[...] = jnp.dot(d2[...], w2ad_t[...], preferred_element_type=f32)
    w2r[...] = jnp.dot(v_r2[...], w2ar_t[...], preferred_element_type=f32)
    u2[...] = jnp.dot(v_beta2[...], w2ab_t[...], preferred_element_type=f32)


def _step_body(c3, d3, r3,
               g_gi, w_gir, b_ih2, w_hh_t, b_hh2,
               w1a_t, b1a2, w1b_t, b1b2, w1c2, b1c2,
               g2, w2r, u2, b2a2, w2b_t, b2b2, w2c2, b2c2, d1,
               alpha_o, beta_o, gamma_o, h_o,
               h_scr, c_scr, gi_scr, m1b_scr, h8_scr, beta8_scr):
    f32 = jnp.float32
    g = pl.program_id(0)

    @pl.when(g == 0)
    def _():
        h_scr[...] = jnp.zeros_like(h_scr)
        c_scr[...] = jnp.zeros_like(c_scr)

    d = d3[0]
    r = r3[0]
    iota_d = jax.lax.broadcasted_iota(jnp.int32, (UB, NUM_D), 1)
    iota_c = jax.lax.broadcasted_iota(jnp.int32, (B, NUM_C), 1)

    oh_d = (d == iota_d).astype(f32)
    gi_scr[...] = (jnp.dot(oh_d, g_gi[...], preferred_element_type=f32)
                   + r * w_gir[...] + b_ih2[...])
    m1b_scr[...] = (jnp.dot(oh_d, g2[...], preferred_element_type=f32)
                    + r * w2r[...] + b2a2[...])
    gamma_o[0] = jnp.dot(oh_d, d1[...], preferred_element_type=f32)

    h = h_scr[...]
    cm = c_scr[...]
    for j in range(U):
        c = c3[0, j * B:(j + 1) * B]
        oh_c = c == iota_c

        gi = gi_scr[j * B:(j + 1) * B, :]
        gh = jnp.dot(h, w_hh_t[...], preferred_element_type=f32) + b_hh2[...]
        rg = jax.nn.sigmoid(gi[:, :DIM_V] + gh[:, :DIM_V])
        z = jax.nn.sigmoid(gi[:, DIM_V:2 * DIM_V] + gh[:, DIM_V:2 * DIM_V])
        n = jnp.tanh(gi[:, 2 * DIM_V:] + rg * gh[:, 2 * DIM_V:])
        h = (1.0 - z) * n + z * h
        h_o[:, j, :] = h
        h8_scr[j * B:(j + 1) * B, :] = h

        bp = jnp.sum(jnp.where(oh_c, cm, 0.0), axis=1, keepdims=True)
        m1 = jnp.maximum(m1b_scr[j * B:(j + 1) * B, :] + bp * u2[...], 0.0)
        m2 = jnp.maximum(jnp.dot(m1, w2b_t[...], preferred_element_type=f32)
                         + b2b2[...], 0.0)
        new_c = jnp.sum(m2 * w2c2[...], axis=1, keepdims=True) + b2c2[...]
        beta8_scr[j * B:(j + 1) * B, :] = new_c
        cm = jnp.where(oh_c, new_c, cm)
    h_scr[...] = h
    c_scr[...] = cm

    h8 = h8_scr[...]
    a1 = jnp.maximum(jnp.dot(h8, w1a_t[...], preferred_element_type=f32)
                     + b1a2[...], 0.0)
    a2 = jnp.maximum(jnp.dot(a1, w1b_t[...], preferred_element_type=f32)
                     + b1b2[...], 0.0)
    alpha_o[0] = jnp.sum(a2 * w1c2[...], axis=1, keepdims=True) + b1c2[...]
    beta_o[0] = beta8_scr[...]


def _sc_fill_body(cbc_hbm, bbc_hbm, out_hbm,
                  c_row, b_row, slab_a, slab_b, sem_a, sem_b):
    f32 = jnp.float32
    wid = lax.axis_index("s") * _NC + lax.axis_index("c")
    lane = lax.iota(jnp.int32, _L)
    zeros16 = jnp.zeros((_L,), f32)
    n_chunks = NUM_C // _L
    npair = (S - U) // (2 * U)

    def drain(slab, sem):
        pltpu.make_async_copy(slab, out_hbm.at[pl.ds(0, U * NUM_C)], sem).wait()

    def fill_row(slab, jr, prev, pr, cj, bj):
        for i in range(n_chunks):
            pv = prev[pl.ds(pr * NUM_C + i * _L, _L)]
            slab[pl.ds(jr * NUM_C + i * _L, _L)] = jnp.where(
                lane + (_L * i) == cj, bj, pv)

    def row_body(rr, _):
        b = wid * _RPW + rr
        pltpu.sync_copy(cbc_hbm.at[b], c_row)
        pltpu.sync_copy(bbc_hbm.at[b], b_row)
        for i in range(n_chunks):
            slab_b[pl.ds((U - 1) * NUM_C + i * _L, _L)] = zeros16
        row_base = b * (S * NUM_C)

        def pair_body(p, _):
            @pl.when(p > 0)
            def _():
                drain(slab_a, sem_a)
                drain(slab_b, sem_b)
            for j in range(2 * U):
                slab, jr = (slab_a, j) if j < U else (slab_b, j - U)
                prev, pr = (slab_b, U - 1) if j == 0 else (
                    (slab_a, j - 1) if j <= U else (slab_b, j - U - 1))
                off = (p * 2 * U + j) * _L
                cj = c_row[pl.ds(off, _L)]
                bj = b_row[pl.ds(off, _L)]
                fill_row(slab, jr, prev, pr, cj, bj)
                if j == U - 1:
                    pltpu.async_copy(
                        slab_a,
                        out_hbm.at[pl.ds(row_base + p * 2 * U * NUM_C,
                                         U * NUM_C)],
                        sem_a)
            pltpu.async_copy(
                slab_b,
                out_hbm.at[pl.ds(row_base + (p * 2 + 1) * U * NUM_C,
                                 U * NUM_C)],
                sem_b)
            return 0

        lax.fori_loop(0, npair, pair_body, 0)

        drain(slab_a, sem_a)
        for j in range(U):
            prev, pr = (slab_b, U - 1) if j == 0 else (slab_a, j - 1)
            off = ((S - U) + j) * _L
            cj = c_row[pl.ds(off, _L)]
            bj = b_row[pl.ds(off, _L)]
            fill_row(slab_a, j, prev, pr, cj, bj)
        pltpu.async_copy(
            slab_a,
            out_hbm.at[pl.ds(row_base + (S - U) * NUM_C, U * NUM_C)],
            sem_a)
        drain(slab_a, sem_a)
        drain(slab_b, sem_b)
        return 0

    lax.fori_loop(0, _RPW, row_body, 0)


def _sc_fill(c_bc, beta_bc):
    f32 = jnp.float32
    mesh = plsc.VectorSubcoreMesh(core_axis_name="c", subcore_axis_name="s")
    k = functools.partial(
        pl.kernel,
        mesh=mesh,
        out_type=jax.ShapeDtypeStruct((B * S * NUM_C,), f32),
        scratch_types=[
            pltpu.VMEM((S * _L,), jnp.int32),
            pltpu.VMEM((S * _L,), f32),
            pltpu.VMEM((U * NUM_C,), f32),
            pltpu.VMEM((U * NUM_C,), f32),
            pltpu.SemaphoreType.DMA,
            pltpu.SemaphoreType.DMA,
        ],
    )(_sc_fill_body)
    return k(c_bc, beta_bc)


def kernel(c_seq, d_seq, r_seq, D1, D2, v_r, v_beta, W_ih, W_hh, b_ih, b_hh,
           W1a, b1a, W1b, b1b, W1c, b1c, W2a, b2a, W2b, b2b, W2c, b2c):
    f32 = jnp.float32
    c_i32 = c_seq.astype(jnp.int32)
    c3 = c_i32.T.reshape(S // U, UB, 1)
    d3 = d_seq.astype(jnp.int32).T.reshape(S // U, UB, 1)
    r3 = r_seq.T.reshape(S // U, UB, 1)
    v_r2 = v_r.reshape(1, DIM_V)
    v_beta2 = v_beta.reshape(1, DIM_V)

    g_gi, w_gir, g2, w2r, u2 = pl.pallas_call(
        _prep_body,
        out_shape=[
            jax.ShapeDtypeStruct((NUM_D, 3 * DIM_V), f32),
            jax.ShapeDtypeStruct((1, 3 * DIM_V), f32),
            jax.ShapeDtypeStruct((NUM_D, DIM_V), f32),
            jax.ShapeDtypeStruct((1, DIM_V), f32),
            jax.ShapeDtypeStruct((1, DIM_V), f32),
        ],
    )(D2, W_ih[:, :DIM_V].T, W_ih[:, DIM_V:].T, v_r2,
      W2a[:, DIM_V:2 * DIM_V].T, W2a[:, 2 * DIM_V:].T, W2a[:, :DIM_V].T,
      v_beta2)

    full = lambda shape: pl.BlockSpec(shape, lambda g: tuple(0 for _ in shape))
    in_specs = [
        pl.BlockSpec((1, UB, 1), lambda g: (g, 0, 0)),
        pl.BlockSpec((1, UB, 1), lambda g: (g, 0, 0)),
        pl.BlockSpec((1, UB, 1), lambda g: (g, 0, 0)),
        full((NUM_D, 3 * DIM_V)),
        full((1, 3 * DIM_V)),
        full((1, 3 * DIM_V)),
        full((DIM_V, 3 * DIM_V)),
        full((1, 3 * DIM_V)),
        full((DIM_V, DIM_V)),
        full((1, DIM_V)),
        full((DIM_V, DIM_V)),
        full((1, DIM_V)),
        full((1, DIM_V)),
        full((1, 1)),
        full((NUM_D, DIM_V)),
        full((1, DIM_V)),
        full((1, DIM_V)),
        full((1, DIM_V)),
        full((DIM_V, DIM_V)),
        full((1, DIM_V)),
        full((1, DIM_V)),
        full((1, 1)),
        full((NUM_D, 1)),
    ]
    out_specs = [
        pl.BlockSpec((1, UB, 1), lambda g: (g, 0, 0)),
        pl.BlockSpec((1, UB, 1), lambda g: (g, 0, 0)),
        pl.BlockSpec((1, UB, 1), lambda g: (g, 0, 0)),
        pl.BlockSpec((B, U, DIM_V), lambda g: (0, g, 0)),
    ]
    out_shape = [
        jax.ShapeDtypeStruct((S // U, UB, 1), f32),
        jax.ShapeDtypeStruct((S // U, UB, 1), f32),
        jax.ShapeDtypeStruct((S // U, UB, 1), f32),
        jax.ShapeDtypeStruct((B, S, DIM_V), f32),
    ]
    alpha, beta, gamma, h_seq = pl.pallas_call(
        _step_body,
        grid=(S // U,),
        in_specs=in_specs,
        out_specs=out_specs,
        out_shape=out_shape,
        scratch_shapes=[
            pltpu.VMEM((B, DIM_V), f32),
            pltpu.VMEM((B, NUM_C), f32),
            pltpu.VMEM((UB, 3 * DIM_V), f32),
            pltpu.VMEM((UB, DIM_V), f32),
            pltpu.VMEM((UB, DIM_V), f32),
            pltpu.VMEM((UB, 1), f32),
        ],
    )(c3, d3, r3,
      g_gi, w_gir, b_ih.reshape(1, -1), W_hh.T, b_hh.reshape(1, -1),
      W1a.T, b1a.reshape(1, -1), W1b.T, b1b.reshape(1, -1),
      W1c.reshape(1, DIM_V), b1c.reshape(1, 1),
      g2, w2r, u2, b2a.reshape(1, -1), W2b.T, b2b.reshape(1, -1),
      W2c.reshape(1, DIM_V), b2c.reshape(1, 1), D1)
    sb = lambda x: x.reshape(S, B).T
    beta_bs = sb(beta)
    c_bc = jnp.broadcast_to(c_i32[:, :, None], (B, S, _L)).reshape(B, S * _L)
    b_bc = jnp.broadcast_to(beta_bs[:, :, None], (B, S, _L)).reshape(B, S * _L)
    c_flat = _sc_fill(c_bc, b_bc)
    return (sb(alpha), beta_bs, sb(gamma), h_seq,
            c_flat.reshape(B, S, NUM_C, 1))
